# baseline (speedup 1.0000x reference)
import jax
import jax.numpy as jnp
from jax import lax
from jax.experimental import pallas as pl
from jax.experimental.pallas import tpu as pltpu
from jax.experimental.pallas import tpu_sc as plsc

NC = 2
NS = 16
NW = NC * NS

S = 10000
SP = 10240
N = 320000
D = 128

C = 80
RPW = N // NW
NIT = RPW // C
RPS = SP // NS
NB = 3


def _sc_body(v_hbm, b_hbm, part_hbm, vbuf_a, vbuf_b, vbuf_c, ibuf, acc,
             sem_in, sem_ib, sem_z, sem_sc):
  c = lax.axis_index("c")
  s = lax.axis_index("s")
  wid = s * NC + c
  r0 = wid * RPW
  bufs = [vbuf_a, vbuf_b, vbuf_c]

  for g in range(NB - 1):
    pltpu.async_copy(v_hbm.at[pl.ds(r0 + g * C, C)], bufs[g], sem_in)
  d_ibuf = pltpu.async_copy(b_hbm.at[wid], ibuf, sem_ib)

  zeros16 = jnp.zeros((16,), jnp.float32)
  def zrow(i, _):
    for j in range(D // 16):
      vbuf_c[i, pl.ds(j * 16, 16)] = zeros16
    return 0
  lax.fori_loop(0, C, zrow, 0)
  zdescs = [
      pltpu.async_copy(vbuf_c, acc.at[pl.ds(s * RPS + t * C, C)], sem_z)
      for t in range(RPS // C)
  ]
  for d in zdescs:
    d.wait()
  d_ibuf.wait()
  plsc.subcore_barrier()

  def drain_sc(buf):
    pltpu.make_async_copy(v_hbm.at[pl.ds(0, C)], buf, sem_sc).wait()

  def step(g, cur, reuse, ingest=True):
    drain_sc(reuse)
    if ingest:
      pltpu.async_copy(v_hbm.at[pl.ds(r0 + (g + NB - 1) * C, C)], reuse,
                       sem_in)
    pltpu.make_async_copy(v_hbm.at[pl.ds(0, C)], cur, sem_in).wait()
    pltpu.async_copy(cur, acc.at[ibuf.at[g]], sem_sc, add=True)

  pltpu.async_copy(v_hbm.at[pl.ds(r0 + 2 * C, C)], vbuf_c, sem_in)
  pltpu.make_async_copy(v_hbm.at[pl.ds(0, C)], vbuf_a, sem_in).wait()
  pltpu.async_copy(vbuf_a, acc.at[ibuf.at[0]], sem_sc, add=True)
  def body(i, _):
    g = 1 + 3 * i
    step(g, vbuf_b, vbuf_a)
    step(g + 1, vbuf_c, vbuf_b)
    step(g + 2, vbuf_a, vbuf_c)
    return 0
  lax.fori_loop(0, 40, body, 0)
  step(121, bufs[121 % NB], bufs[120 % NB])
  step(122, bufs[122 % NB], bufs[121 % NB])
  step(123, bufs[123 % NB], bufs[122 % NB], ingest=False)
  step(124, bufs[124 % NB], bufs[123 % NB], ingest=False)
  drain_sc(vbuf_a)
  plsc.subcore_barrier()

  pltpu.sync_copy(acc.at[pl.ds(s * RPS, RPS)],
                  part_hbm.at[c, pl.ds(s * RPS, RPS)])


_sc_scatter = pl.kernel(
    _sc_body,
    out_type=jax.ShapeDtypeStruct((NC, SP, D), jnp.float32),
    mesh=plsc.VectorSubcoreMesh(core_axis_name="c", subcore_axis_name="s"),
    scratch_types=[
        pltpu.VMEM((C, D), jnp.float32),
        pltpu.VMEM((C, D), jnp.float32),
        pltpu.VMEM((C, D), jnp.float32),
        pltpu.VMEM((NIT, C), jnp.int32),
        pltpu.VMEM_SHARED((SP, D), jnp.float32),
        pltpu.SemaphoreType.DMA,
        pltpu.SemaphoreType.DMA,
        pltpu.SemaphoreType.DMA,
        pltpu.SemaphoreType.DMA,
    ],
)


def _combine_body(u_ref, p_ref, o_ref):
  o_ref[...] = u_ref[...] + p_ref[0] + p_ref[1]


_combine = pl.pallas_call(
    _combine_body,
    grid=(10,),
    in_specs=[
        pl.BlockSpec((1000, D), lambda i: (i, 0)),
        pl.BlockSpec((NC, 1000, D), lambda i: (0, i, 0)),
    ],
    out_specs=pl.BlockSpec((1000, D), lambda i: (i, 0)),
    out_shape=jax.ShapeDtypeStruct((S, D), jnp.float32),
)


@jax.jit
def kernel(u, v, batch):
  b3 = batch.reshape(NW, NIT, C)
  parts = _sc_scatter(v, b3)
  return _combine(u, parts)


# speedup vs baseline: 1.0398x; 1.0398x over previous
import jax
import jax.numpy as jnp
from jax import lax
from jax.experimental import pallas as pl
from jax.experimental.pallas import tpu as pltpu
from jax.experimental.pallas import tpu_sc as plsc

NC = 2
NS = 16
S = 10000
HALF = S // 2
N = 320000
D = 128

C = 128
NTOT = N // C
NWIN = N // 16
TRASH = HALF
AR = HALF + 8
USML = HALF // NS
USBIG = HALF - 15 * USML


def _sc_body(u_hbm, v_hbm, b_hbm, out_hbm, vbuf_a, vbuf_b, vbuf_c,
             iring, sbuf_v, acc, sem_in, sem_ix, sem_u, sem_sc):
  c = lax.axis_index("c")
  s = lax.axis_index("s")
  bufs = [vbuf_a, vbuf_b, vbuf_c]

  @pl.when(s < NS - 1)
  def _():
    pltpu.async_copy(u_hbm.at[pl.ds(c * HALF + s * USML, USML)],
                     acc.at[pl.ds(s * USML, USML)], sem_u)
  @pl.when(s == NS - 1)
  def _():
    pltpu.async_copy(u_hbm.at[pl.ds(c * HALF + 15 * USML, USBIG)],
                     acc.at[pl.ds(15 * USML, USBIG)], sem_u)

  def bs_round(_, carry):
    lo, hi = carry
    done = lo >= hi
    wi = jnp.minimum((lo + hi) // 2, N // 8 - 1)
    p = 8 * wi
    wstart = jnp.minimum(p, N - 16)
    pltpu.sync_copy(b_hbm.at[pl.ds(wstart, 16)], sbuf_v)
    vec = sbuf_v[...]
    val = jnp.where(p == wstart, vec[0], vec[8])
    pred = val >= HALF
    return (jnp.where(done, lo, jnp.where(pred, lo, wi + 1)),
            jnp.where(done, hi, jnp.where(pred, wi, hi)))
  lo8, _ = lax.fori_loop(0, 16, bs_round, (jnp.int32(0), jnp.int32(N // 8)))
  rstar = 8 * lo8

  k_split = (rstar + (C - 1)) // C
  start = jnp.where(c == 0, 0, jnp.maximum(k_split - 1, 0))
  end = jnp.where(c == 0, k_split, NTOT)
  t_cnt = jnp.maximum((end - start - s + (NS - 1)) // NS, 0)
  base = start + s

  @pl.when(s < NS - 1)
  def _():
    pltpu.make_async_copy(u_hbm.at[pl.ds(0, USML)], acc.at[pl.ds(0, USML)],
                          sem_u).wait()
  @pl.when(s == NS - 1)
  def _():
    pltpu.make_async_copy(u_hbm.at[pl.ds(0, USBIG)], acc.at[pl.ds(0, USBIG)],
                          sem_u).wait()
  plsc.subcore_barrier()

  lo_vec = jnp.full((16,), 0, jnp.int32)
  hi_vec = jnp.full((16,), HALF, jnp.int32)
  trash16 = jnp.full((16,), TRASH, jnp.int32)

  def ingest(j, b):
    k = base + NS * j
    pltpu.async_copy(v_hbm.at[pl.ds(k * C, C)], bufs[b], sem_in)
    pltpu.async_copy(b_hbm.at[pl.ds(k * C, C)], iring.at[b], sem_ix)

  @pl.when(t_cnt > 0)
  def _():
    ingest(0, 0)
  @pl.when(t_cnt > 1)
  def _():
    ingest(1, 1)

  cbase = c * HALF

  def step(j, b):
    @pl.when(j + 2 < t_cnt)
    def _():
      ingest(j + 2, (b + 2) % 3)
    pltpu.make_async_copy(v_hbm.at[pl.ds(0, C)], bufs[b], sem_in).wait()
    pltpu.make_async_copy(b_hbm.at[pl.ds(0, C)], iring.at[b], sem_ix).wait()
    for q in range(C // 16):
      w = iring[b, pl.ds(q * 16, 16)] - cbase
      bad = (w < lo_vec) | (w >= hi_vec)
      iring[b, pl.ds(q * 16, 16)] = jnp.where(bad, trash16, w)
    pltpu.async_copy(bufs[b], acc.at[iring.at[b]], sem_sc, add=True)

  def body(j, _):
    @pl.when(j >= 1)
    def _():
      pltpu.make_async_copy(v_hbm.at[pl.ds(0, C)], vbuf_a, sem_sc).wait()
    for b in range(3):
      @pl.when(j % 3 == b)
      def _():
        step(j, b)
    return 0
  lax.fori_loop(0, t_cnt, body, 0)
  @pl.when(t_cnt > 0)
  def _():
    pltpu.make_async_copy(v_hbm.at[pl.ds(0, C)], vbuf_a, sem_sc).wait()
  plsc.subcore_barrier()

  @pl.when(s < NS - 1)
  def _():
    pltpu.sync_copy(acc.at[pl.ds(s * USML, USML)],
                    out_hbm.at[pl.ds(c * HALF + s * USML, USML)])
  @pl.when(s == NS - 1)
  def _():
    pltpu.sync_copy(acc.at[pl.ds(15 * USML, USBIG)],
                    out_hbm.at[pl.ds(c * HALF + 15 * USML, USBIG)])


_sc_kernel = pl.kernel(
    _sc_body,
    out_type=jax.ShapeDtypeStruct((S, D), jnp.float32),
    mesh=plsc.VectorSubcoreMesh(core_axis_name="c", subcore_axis_name="s"),
    scratch_types=[
        pltpu.VMEM((C, D), jnp.float32),
        pltpu.VMEM((C, D), jnp.float32),
        pltpu.VMEM((C, D), jnp.float32),
        pltpu.VMEM((3, C), jnp.int32),
        pltpu.VMEM((16,), jnp.int32),
        pltpu.VMEM_SHARED((AR, D), jnp.float32),
        pltpu.SemaphoreType.DMA,
        pltpu.SemaphoreType.DMA,
        pltpu.SemaphoreType.DMA,
        pltpu.SemaphoreType.DMA,
    ],
)


@jax.jit
def kernel(u, v, batch):
  return _sc_kernel(u, v, batch)


# speedup vs baseline: 1.0711x; 1.0301x over previous
import jax
import jax.numpy as jnp
from jax import lax
from jax.experimental import pallas as pl
from jax.experimental.pallas import tpu as pltpu
from jax.experimental.pallas import tpu_sc as plsc

NC = 2
NS = 16
S = 10000
HALF = S // 2
N = 320000
D = 128

C = 64
NTOT = N // C
NWIN = N // 16
TRASH = HALF
AR = HALF + 8
USML = HALF // NS
USBIG = HALF - 15 * USML


def _sc_body(u_hbm, v_hbm, b_hbm, out_hbm, vbuf_a, vbuf_b, vbuf_c,
             iring, sbuf_v, acc, sem_in, sem_ix, sem_u, sem_sc):
  c = lax.axis_index("c")
  s = lax.axis_index("s")
  bufs = [vbuf_a, vbuf_b, vbuf_c]

  @pl.when(s < NS - 1)
  def _():
    pltpu.async_copy(u_hbm.at[pl.ds(c * HALF + s * USML, USML)],
                     acc.at[pl.ds(s * USML, USML)], sem_u)
  @pl.when(s == NS - 1)
  def _():
    pltpu.async_copy(u_hbm.at[pl.ds(c * HALF + 15 * USML, USBIG)],
                     acc.at[pl.ds(15 * USML, USBIG)], sem_u)

  def bs_round(_, carry):
    lo, hi = carry
    done = lo >= hi
    wi = jnp.minimum((lo + hi) // 2, N // 8 - 1)
    p = 8 * wi
    wstart = jnp.minimum(p, N - 16)
    pltpu.sync_copy(b_hbm.at[pl.ds(wstart, 16)], sbuf_v)
    vec = sbuf_v[...]
    val = jnp.where(p == wstart, vec[0], vec[8])
    pred = val >= HALF
    return (jnp.where(done, lo, jnp.where(pred, lo, wi + 1)),
            jnp.where(done, hi, jnp.where(pred, wi, hi)))
  lo8, _ = lax.fori_loop(0, 16, bs_round, (jnp.int32(0), jnp.int32(N // 8)))
  rstar = 8 * lo8

  k_split = (rstar + (C - 1)) // C
  start = jnp.where(c == 0, 0, jnp.maximum(k_split - 1, 0))
  end = jnp.where(c == 0, k_split, NTOT)
  t_cnt = jnp.maximum((end - start - s + (NS - 1)) // NS, 0)
  base = start + s

  @pl.when(s < NS - 1)
  def _():
    pltpu.make_async_copy(u_hbm.at[pl.ds(0, USML)], acc.at[pl.ds(0, USML)],
                          sem_u).wait()
  @pl.when(s == NS - 1)
  def _():
    pltpu.make_async_copy(u_hbm.at[pl.ds(0, USBIG)], acc.at[pl.ds(0, USBIG)],
                          sem_u).wait()
  plsc.subcore_barrier()

  lo_vec = jnp.full((16,), 0, jnp.int32)
  hi_vec = jnp.full((16,), HALF, jnp.int32)
  trash16 = jnp.full((16,), TRASH, jnp.int32)

  def ingest(j, b):
    k = base + NS * j
    pltpu.async_copy(v_hbm.at[pl.ds(k * C, C)], bufs[b], sem_in)
    pltpu.async_copy(b_hbm.at[pl.ds(k * C, C)], iring.at[b], sem_ix)

  @pl.when(t_cnt > 0)
  def _():
    ingest(0, 0)
  @pl.when(t_cnt > 1)
  def _():
    ingest(1, 1)

  cbase = c * HALF

  def step(j, b):
    @pl.when(j + 2 < t_cnt)
    def _():
      ingest(j + 2, (b + 2) % 3)
    pltpu.make_async_copy(v_hbm.at[pl.ds(0, C)], bufs[b], sem_in).wait()
    pltpu.make_async_copy(b_hbm.at[pl.ds(0, C)], iring.at[b], sem_ix).wait()
    for q in range(C // 16):
      w = iring[b, pl.ds(q * 16, 16)] - cbase
      bad = (w < lo_vec) | (w >= hi_vec)
      iring[b, pl.ds(q * 16, 16)] = jnp.where(bad, trash16, w)
    pltpu.async_copy(bufs[b], acc.at[iring.at[b]], sem_sc, add=True)

  def body(j, _):
    @pl.when(j >= 1)
    def _():
      pltpu.make_async_copy(v_hbm.at[pl.ds(0, C)], vbuf_a, sem_sc).wait()
    for b in range(3):
      @pl.when(j % 3 == b)
      def _():
        step(j, b)
    return 0
  lax.fori_loop(0, t_cnt, body, 0)
  @pl.when(t_cnt > 0)
  def _():
    pltpu.make_async_copy(v_hbm.at[pl.ds(0, C)], vbuf_a, sem_sc).wait()
  plsc.subcore_barrier()

  @pl.when(s < NS - 1)
  def _():
    pltpu.sync_copy(acc.at[pl.ds(s * USML, USML)],
                    out_hbm.at[pl.ds(c * HALF + s * USML, USML)])
  @pl.when(s == NS - 1)
  def _():
    pltpu.sync_copy(acc.at[pl.ds(15 * USML, USBIG)],
                    out_hbm.at[pl.ds(c * HALF + 15 * USML, USBIG)])


_sc_kernel = pl.kernel(
    _sc_body,
    out_type=jax.ShapeDtypeStruct((S, D), jnp.float32),
    mesh=plsc.VectorSubcoreMesh(core_axis_name="c", subcore_axis_name="s"),
    scratch_types=[
        pltpu.VMEM((C, D), jnp.float32),
        pltpu.VMEM((C, D), jnp.float32),
        pltpu.VMEM((C, D), jnp.float32),
        pltpu.VMEM((3, C), jnp.int32),
        pltpu.VMEM((16,), jnp.int32),
        pltpu.VMEM_SHARED((AR, D), jnp.float32),
        pltpu.SemaphoreType.DMA,
        pltpu.SemaphoreType.DMA,
        pltpu.SemaphoreType.DMA,
        pltpu.SemaphoreType.DMA,
    ],
)


@jax.jit
def kernel(u, v, batch):
  return _sc_kernel(u, v, batch)
